# baseline (speedup 1.0000x reference)
import jax
import jax.numpy as jnp
from jax import lax
from jax.experimental import pallas as pl
from jax.experimental.pallas import tpu as pltpu
from jax.experimental.pallas import tpu_sc as plsc

B = 4096
L = 50
D = 32
NC = 2
NS = 16
NW = NC * NS
BPW = B // NW
CHUNK = 32
NCHUNK = BPW // CHUNK
F32_MAX = 3.4028235e38


def _sc_body(text_ref, ai0_ref, ai1_ref, ai2_ref, emb_ref, cat0_ref, cat1_ref,
             cat2_ref, sum_ref, cat_ref, idx_v, rows_v, cat_idx_v, cat_rows_v,
             sum_v, catsum_v, gsem, csem):
    wid = lax.axis_index("s") * NC + lax.axis_index("c")
    base = wid * BPW

    pltpu.sync_copy(text_ref.at[pl.ds(base, BPW)], idx_v)
    ai_refs = (ai0_ref, ai1_ref, ai2_ref)
    for c in range(3):
        pltpu.sync_copy(ai_refs[c].at[pl.ds(base, BPW)], cat_idx_v.at[c])

    cat_tables = (cat0_ref, cat1_ref, cat2_ref)
    cat_descs = [
        pltpu.async_copy(cat_tables[c].at[cat_idx_v.at[c]], cat_rows_v.at[c], csem)
        for c in range(3)
    ]

    def compute_row(r, chunk_base):
        a0 = [jnp.zeros((16,), jnp.float32) for _ in range(4)]
        a1 = [jnp.zeros((16,), jnp.float32) for _ in range(4)]
        for t in range(L):
            a0[t % 4] = a0[t % 4] + rows_v[r, t, pl.ds(0, 16)]
            a1[t % 4] = a1[t % 4] + rows_v[r, t, pl.ds(16, 16)]
        row = chunk_base + r
        sum_v[row, pl.ds(0, 16)] = (a0[0] + a0[1]) + (a0[2] + a0[3])
        sum_v[row, pl.ds(16, 16)] = (a1[0] + a1[1]) + (a1[2] + a1[3])

    for chunk in range(NCHUNK):
        cb = chunk * CHUNK
        descs = [
            pltpu.async_copy(emb_ref.at[idx_v.at[cb + r]], rows_v.at[r], gsem)
            for r in range(CHUNK)
        ]
        for d in descs:
            d.wait()

        def body(r, carry):
            compute_row(r, cb)
            return carry

        lax.fori_loop(0, CHUNK, body, jnp.int32(0))

    for d in cat_descs:
        d.wait()

    def cat_body(r, carry):
        for h in (0, 16):
            catsum_v[r, pl.ds(h, 16)] = (
                cat_rows_v[0, r, pl.ds(h, 16)]
                + cat_rows_v[1, r, pl.ds(h, 16)]
                + cat_rows_v[2, r, pl.ds(h, 16)]
            )
        return carry

    lax.fori_loop(0, BPW, cat_body, jnp.int32(0))

    pltpu.sync_copy(sum_v, sum_ref.at[pl.ds(base, BPW)])
    pltpu.sync_copy(catsum_v, cat_ref.at[pl.ds(base, BPW)])


@jax.jit
def _sc_pool(encoded_text, ai0, ai1, ai2, emb_table, cat_emb0, cat_emb1,
             cat_emb2):
    mesh = plsc.VectorSubcoreMesh(
        core_axis_name="c", subcore_axis_name="s", num_cores=NC, num_subcores=NS
    )
    return pl.kernel(
        _sc_body,
        out_type=(
            jax.ShapeDtypeStruct((B, D), jnp.float32),
            jax.ShapeDtypeStruct((B, D), jnp.float32),
        ),
        mesh=mesh,
        compiler_params=pltpu.CompilerParams(use_tc_tiling_on_sc=False),
        scratch_types=[
            pltpu.VMEM((BPW, L), jnp.int32),
            pltpu.VMEM((CHUNK, L, D), jnp.float32),
            pltpu.VMEM((3, BPW), jnp.int32),
            pltpu.VMEM((3, BPW, D), jnp.float32),
            pltpu.VMEM((BPW, D), jnp.float32),
            pltpu.VMEM((BPW, D), jnp.float32),
            pltpu.SemaphoreType.DMA,
            pltpu.SemaphoreType.DMA,
        ],
    )(encoded_text, ai0, ai1, ai2, emb_table, cat_emb0, cat_emb1, cat_emb2)


def _head_body(text_ref, sum_ref, cat_ref, w_ref, b_ref, o_ref):
    cnt = jnp.sum((text_ref[...] != 0).astype(jnp.float32), axis=1,
                  keepdims=True)
    x = sum_ref[...] / cnt
    x = jnp.where(x != x, jnp.float32(0.0), x)
    x = jnp.minimum(jnp.maximum(x, -F32_MAX), F32_MAX)
    x = x + cat_ref[...]
    o_ref[...] = (
        jnp.dot(x, w_ref[...], preferred_element_type=jnp.float32) + b_ref[...]
    )


@jax.jit
def _tc_head(encoded_text, x_sum, cat_sum, w, b2d):
    bm = 512
    nc = w.shape[1]
    return pl.pallas_call(
        _head_body,
        grid=(B // bm,),
        in_specs=[
            pl.BlockSpec((bm, L), lambda i: (i, 0)),
            pl.BlockSpec((bm, D), lambda i: (i, 0)),
            pl.BlockSpec((bm, D), lambda i: (i, 0)),
            pl.BlockSpec((D, nc), lambda i: (0, 0)),
            pl.BlockSpec((1, nc), lambda i: (0, 0)),
        ],
        out_specs=pl.BlockSpec((bm, nc), lambda i: (i, 0)),
        out_shape=jax.ShapeDtypeStruct((B, nc), jnp.float32),
    )(encoded_text, x_sum, cat_sum, w, b2d)


def kernel(encoded_text, additional_inputs, emb_table, cat_emb0, cat_emb1,
           cat_emb2, fc_w, fc_b):
    text = encoded_text.astype(jnp.int32)
    ai = additional_inputs.astype(jnp.int32)
    x_sum, cat_sum = _sc_pool(text, ai[:, 0], ai[:, 1], ai[:, 2], emb_table,
                              cat_emb0, cat_emb1, cat_emb2)
    return _tc_head(text, x_sum, cat_sum, fc_w, fc_b.reshape(1, -1))
